# baseline (speedup 1.0000x reference)
import jax
import jax.numpy as jnp
from jax import lax
from jax.experimental import pallas as pl
from jax.experimental.pallas import tpu as pltpu


def _round_up(x, m):
    return (x + m - 1) // m * m


def _neumf_kernel(uid_ref, iid_ref,
                  utab_ref, itab_ref,
                  w1u_ref, w1i_ref, b1_ref, w2_ref, b2_ref,
                  wng_ref, wnm_ref, bn_ref,
                  out_ref, ue_s, ie_s):
    TB = out_ref.shape[1]
    base = pl.program_id(0) * TB
    CH = 128 if TB % 128 == 0 else 8

    def chunk(c, carry):
        cb = pl.multiple_of(c * CH, CH)
        for k in range(CH):
            ue_s[pl.ds(cb + k, 1), :] = utab_ref[pl.ds(uid_ref[base + cb + k], 1), :]
            ie_s[pl.ds(cb + k, 1), :] = itab_ref[pl.ds(iid_ref[base + cb + k], 1), :]
        return carry

    lax.fori_loop(0, TB // CH, chunk, 0)

    ue = ue_s[...]
    ie = ie_s[...]
    prod = ue * ie

    h = (jnp.dot(ue, w1u_ref[...], preferred_element_type=jnp.float32)
         + jnp.dot(ie, w1i_ref[...], preferred_element_type=jnp.float32)
         + b1_ref[...])
    h = jnp.maximum(h, 0.0)
    h = jnp.maximum(
        jnp.dot(h, w2_ref[...], preferred_element_type=jnp.float32) + b2_ref[...],
        0.0)

    logit = (lax.dot_general(wng_ref[...], prod, (((1,), (1,)), ((), ())),
                             preferred_element_type=jnp.float32)
             + lax.dot_general(wnm_ref[...], h, (((1,), (1,)), ((), ())),
                               preferred_element_type=jnp.float32)
             + bn_ref[...])

    out_ref[...] = 1.0 / (1.0 + jnp.exp(-logit))


def kernel(batch, gmf_user, gmf_item, mlp_user, mlp_item, w1, b1, w2, b2, wn, bn):
    uid = batch[:, 0].astype(jnp.int32)
    iid = batch[:, 1].astype(jnp.int32)
    B = batch.shape[0]
    U, Eg = gmf_user.shape
    I = gmf_item.shape[0]
    Em = mlp_user.shape[1]
    L1 = w1.shape[1]
    L2 = w2.shape[1]
    F = Eg + Em
    assert F % 128 == 0, "fused row width must be lane-aligned"

    f32 = jnp.float32
    def _fuse(g, m):
        return (jnp.pad(g.astype(f32), ((0, 0), (0, F - Eg)))
                + jnp.pad(m.astype(f32), ((0, 0), (Eg, F - Eg - Em))))

    user_tab = _fuse(gmf_user, mlp_user)
    item_tab = _fuse(gmf_item, mlp_item)

    w1u = jnp.zeros((F, L1), f32).at[Eg:Eg + Em, :].set(w1[:Em].astype(f32))
    w1i = jnp.zeros((F, L1), f32).at[Eg:Eg + Em, :].set(w1[Em:].astype(f32))
    wn_g = jnp.zeros((1, F), f32).at[:, :Eg].set(wn[:, :Eg].astype(f32))
    wn_m = wn[:, Eg:].astype(f32)
    w2_f = w2.astype(f32)
    b1_2d = b1.reshape(1, L1).astype(f32)
    b2_2d = b2.reshape(1, L2).astype(f32)
    bn_2d = bn.reshape(1, 1).astype(f32)

    if B <= 1024:
        TB = _round_up(B, 8)
    else:
        TB = min(4096, 1024 * max(1, B // 2048))
    Bp = _round_up(B, TB)
    if Bp != B:
        pad = jnp.zeros((Bp - B,), jnp.int32)
        uid = jnp.concatenate([uid, pad])
        iid = jnp.concatenate([iid, pad])
    nt = Bp // TB

    vmem_need = 4 * (2 * (user_tab.size + item_tab.size) + 4 * TB * F) + (8 << 20)
    vmem_limit = int(min(max(vmem_need, 32 << 20), 64 << 20))
    compiler_params = pltpu.CompilerParams(
        dimension_semantics=("arbitrary",),
        vmem_limit_bytes=vmem_limit,
    )
    out_shape = jax.ShapeDtypeStruct((1, Bp), jnp.float32)

    const = lambda t, us, js: (0, 0)
    grid_spec = pltpu.PrefetchScalarGridSpec(
        num_scalar_prefetch=2,
        grid=(nt,),
        in_specs=[
            pl.BlockSpec((U, F), const),
            pl.BlockSpec((I, F), const),
            pl.BlockSpec((F, L1), const),
            pl.BlockSpec((F, L1), const),
            pl.BlockSpec((1, L1), const),
            pl.BlockSpec((L1, L2), const),
            pl.BlockSpec((1, L2), const),
            pl.BlockSpec((1, F), const),
            pl.BlockSpec((1, L2), const),
            pl.BlockSpec((1, 1), const),
        ],
        out_specs=pl.BlockSpec((1, TB), lambda t, us, js: (0, t)),
        scratch_shapes=[
            pltpu.VMEM((TB, F), jnp.float32),
            pltpu.VMEM((TB, F), jnp.float32),
        ],
    )
    out = pl.pallas_call(
        _neumf_kernel,
        out_shape=out_shape,
        grid_spec=grid_spec,
        compiler_params=compiler_params,
    )(uid, iid,
      user_tab, item_tab, w1u, w1i, b1_2d, w2_f, b2_2d, wn_g, wn_m, bn_2d)

    return out[0, :B]


# speedup vs baseline: 1.0134x; 1.0134x over previous
import jax
import jax.numpy as jnp
from jax import lax
from jax.experimental import pallas as pl
from jax.experimental.pallas import tpu as pltpu


def _round_up(x, m):
    return (x + m - 1) // m * m


def _neumf_kernel(uid_ref, iid_ref,
                  utab_ref, itab_ref,
                  w1u_ref, w1i_ref, b1_ref, w2_ref, b2_ref,
                  wng_ref, wnm_ref, bn_ref,
                  out_ref, ue_s, ie_s):
    TB = out_ref.shape[1]
    base = pl.program_id(0) * TB
    CH = 128 if TB % 128 == 0 else 8

    def chunk(c, carry):
        cb = pl.multiple_of(c * CH, CH)
        for k in range(CH):
            ue_s[pl.ds(cb + k, 1), :] = utab_ref[pl.ds(uid_ref[base + cb + k], 1), :]
            ie_s[pl.ds(cb + k, 1), :] = itab_ref[pl.ds(iid_ref[base + cb + k], 1), :]
        return carry

    lax.fori_loop(0, TB // CH, chunk, 0)

    ue = ue_s[...]
    ie = ie_s[...]
    prod = ue * ie

    h = (jnp.dot(ue, w1u_ref[...], preferred_element_type=jnp.float32)
         + jnp.dot(ie, w1i_ref[...], preferred_element_type=jnp.float32)
         + b1_ref[...])
    h = jnp.maximum(h, 0.0)
    h = jnp.maximum(
        jnp.dot(h, w2_ref[...], preferred_element_type=jnp.float32) + b2_ref[...],
        0.0)

    logit = (lax.dot_general(wng_ref[...], prod, (((1,), (1,)), ((), ())),
                             preferred_element_type=jnp.float32)
             + lax.dot_general(wnm_ref[...], h, (((1,), (1,)), ((), ())),
                               preferred_element_type=jnp.float32)
             + bn_ref[...])

    out_ref[...] = 1.0 / (1.0 + jnp.exp(-logit))


def kernel(batch, gmf_user, gmf_item, mlp_user, mlp_item, w1, b1, w2, b2, wn, bn):
    uid = batch[:, 0].astype(jnp.int32)
    iid = batch[:, 1].astype(jnp.int32)
    B = batch.shape[0]
    U, Eg = gmf_user.shape
    I = gmf_item.shape[0]
    Em = mlp_user.shape[1]
    L1 = w1.shape[1]
    L2 = w2.shape[1]
    F = Eg + Em
    assert F % 128 == 0, "fused row width must be lane-aligned"

    f32 = jnp.float32
    def _fuse(g, m):
        return (jnp.pad(g.astype(f32), ((0, 0), (0, F - Eg)))
                + jnp.pad(m.astype(f32), ((0, 0), (Eg, F - Eg - Em))))

    user_tab = _fuse(gmf_user, mlp_user)
    item_tab = _fuse(gmf_item, mlp_item)

    w1u = jnp.zeros((F, L1), f32).at[Eg:Eg + Em, :].set(w1[:Em].astype(f32))
    w1i = jnp.zeros((F, L1), f32).at[Eg:Eg + Em, :].set(w1[Em:].astype(f32))
    wn_g = jnp.zeros((1, F), f32).at[:, :Eg].set(wn[:, :Eg].astype(f32))
    wn_m = wn[:, Eg:].astype(f32)
    w2_f = w2.astype(f32)
    b1_2d = b1.reshape(1, L1).astype(f32)
    b2_2d = b2.reshape(1, L2).astype(f32)
    bn_2d = bn.reshape(1, 1).astype(f32)

    if B <= 1024:
        TB = _round_up(B, 8)
    else:
        TB = min(8192, 1024 * max(1, B // 2048))
    Bp = _round_up(B, TB)
    if Bp != B:
        pad = jnp.zeros((Bp - B,), jnp.int32)
        uid = jnp.concatenate([uid, pad])
        iid = jnp.concatenate([iid, pad])
    nt = Bp // TB

    vmem_need = 4 * (2 * (user_tab.size + item_tab.size) + 4 * TB * F) + (8 << 20)
    vmem_limit = int(min(max(vmem_need, 32 << 20), 64 << 20))
    compiler_params = pltpu.CompilerParams(
        dimension_semantics=("arbitrary",),
        vmem_limit_bytes=vmem_limit,
    )
    out_shape = jax.ShapeDtypeStruct((1, Bp), jnp.float32)

    const = lambda t, us, js: (0, 0)
    grid_spec = pltpu.PrefetchScalarGridSpec(
        num_scalar_prefetch=2,
        grid=(nt,),
        in_specs=[
            pl.BlockSpec((U, F), const),
            pl.BlockSpec((I, F), const),
            pl.BlockSpec((F, L1), const),
            pl.BlockSpec((F, L1), const),
            pl.BlockSpec((1, L1), const),
            pl.BlockSpec((L1, L2), const),
            pl.BlockSpec((1, L2), const),
            pl.BlockSpec((1, F), const),
            pl.BlockSpec((1, L2), const),
            pl.BlockSpec((1, 1), const),
        ],
        out_specs=pl.BlockSpec((1, TB), lambda t, us, js: (0, t)),
        scratch_shapes=[
            pltpu.VMEM((TB, F), jnp.float32),
            pltpu.VMEM((TB, F), jnp.float32),
        ],
    )
    out = pl.pallas_call(
        _neumf_kernel,
        out_shape=out_shape,
        grid_spec=grid_spec,
        compiler_params=compiler_params,
    )(uid, iid,
      user_tab, item_tab, w1u, w1i, b1_2d, w2_f, b2_2d, wn_g, wn_m, bn_2d)

    return out[0, :B]
